# baseline (speedup 1.0000x reference)
import functools

import jax
import jax.numpy as jnp
from jax.experimental import pallas as pl
from jax.experimental.pallas import tpu as pltpu

B = 32
C = 768
HW = 1024
EPS = 1e-10


def _kernel(x_ref, quant_ref, ind_ref, div_ref, perp_ref, qsum_ref, cnt_ref):
    b = pl.program_id(0)

    x = x_ref[0]
    r = jnp.maximum(x, 0.0)
    s = jnp.sum(r, axis=0, keepdims=True)
    norm = r / (s + EPS)

    m = jnp.max(r, axis=0, keepdims=True)
    ci = jax.lax.broadcasted_iota(jnp.int32, (C, HW), 0)
    idx = jnp.min(jnp.where(r == m, ci, C), axis=0, keepdims=True)

    onehot = (ci == idx).astype(jnp.float32)
    quant_ref[0] = onehot
    ind_ref[0] = idx

    @pl.when(b == 0)
    def _init():
        qsum_ref[...] = jnp.zeros_like(qsum_ref)
        cnt_ref[...] = jnp.zeros_like(cnt_ref)

    qsum_ref[...] += jnp.sum(norm, axis=1, keepdims=True)
    cnt_ref[...] += jnp.sum(onehot, axis=1, keepdims=True)

    @pl.when(b == B - 1)
    def _finish():
        n = float(B * HW)
        q_bar = qsum_ref[...] / n
        div_ref[...] = jnp.mean((q_bar * C - 1.0) ** 2, keepdims=True)
        p = cnt_ref[...] / n
        perp_ref[...] = jnp.exp(-jnp.sum(p * jnp.log(p + 1e-10), keepdims=True))


@jax.jit
def kernel(x):
    xr = x.reshape(B, C, HW)
    quant, ind, div, perp = pl.pallas_call(
        _kernel,
        grid=(B,),
        in_specs=[pl.BlockSpec((1, C, HW), lambda b: (b, 0, 0))],
        out_specs=[
            pl.BlockSpec((1, C, HW), lambda b: (b, 0, 0)),
            pl.BlockSpec((1, 1, HW), lambda b: (b, 0, 0)),
            pl.BlockSpec((1, 1), lambda b: (0, 0)),
            pl.BlockSpec((1, 1), lambda b: (0, 0)),
        ],
        out_shape=[
            jax.ShapeDtypeStruct((B, C, HW), jnp.float32),
            jax.ShapeDtypeStruct((B, 1, HW), jnp.int32),
            jax.ShapeDtypeStruct((1, 1), jnp.float32),
            jax.ShapeDtypeStruct((1, 1), jnp.float32),
        ],
        scratch_shapes=[
            pltpu.VMEM((C, 1), jnp.float32),
            pltpu.VMEM((C, 1), jnp.float32),
        ],
    )(xr)
    quantize = quant.reshape(B, C, 32, 32)
    embed_ind = ind.reshape(B, 32, 32)
    return (quantize, div[0, 0], embed_ind, perp[0, 0])


# speedup vs baseline: 1.0079x; 1.0079x over previous
import functools

import jax
import jax.numpy as jnp
from jax.experimental import pallas as pl
from jax.experimental.pallas import tpu as pltpu

B = 32
C = 768
HW = 1024
EPS = 1e-10


def _kernel(x_ref, quant_ref, ind_ref, div_ref, perp_ref, qsum_ref, cnt_ref):
    b = pl.program_id(0)

    x = x_ref[0]
    r = jnp.maximum(x, 0.0)
    s = jnp.sum(r, axis=0, keepdims=True)
    w = 1.0 / (s + EPS)
    norm = r * w

    m = jnp.max(r, axis=0, keepdims=True)
    ci = jax.lax.broadcasted_iota(jnp.int32, (C, HW), 0)
    idx = jnp.min(jnp.where(r == m, ci, C), axis=0, keepdims=True)

    onehot = (ci == idx).astype(jnp.float32)
    quant_ref[0] = onehot
    ind_ref[0] = idx

    @pl.when(b == 0)
    def _init():
        qsum_ref[...] = jnp.zeros_like(qsum_ref)
        cnt_ref[...] = jnp.zeros_like(cnt_ref)

    qsum_ref[...] += jnp.sum(norm, axis=1, keepdims=True)
    cnt_ref[...] += jnp.sum(onehot, axis=1, keepdims=True)

    @pl.when(b == B - 1)
    def _finish():
        n = float(B * HW)
        q_bar = qsum_ref[...] / n
        div_ref[...] = jnp.mean((q_bar * C - 1.0) ** 2, keepdims=True)
        p = cnt_ref[...] / n
        perp_ref[...] = jnp.exp(-jnp.sum(p * jnp.log(p + 1e-10), keepdims=True))


@jax.jit
def kernel(x):
    xr = x.reshape(B, C, HW)
    quant, ind, div, perp = pl.pallas_call(
        _kernel,
        grid=(B,),
        in_specs=[pl.BlockSpec((1, C, HW), lambda b: (b, 0, 0))],
        out_specs=[
            pl.BlockSpec((1, C, HW), lambda b: (b, 0, 0)),
            pl.BlockSpec((1, 1, HW), lambda b: (b, 0, 0)),
            pl.BlockSpec((1, 1), lambda b: (0, 0)),
            pl.BlockSpec((1, 1), lambda b: (0, 0)),
        ],
        out_shape=[
            jax.ShapeDtypeStruct((B, C, HW), jnp.float32),
            jax.ShapeDtypeStruct((B, 1, HW), jnp.int32),
            jax.ShapeDtypeStruct((1, 1), jnp.float32),
            jax.ShapeDtypeStruct((1, 1), jnp.float32),
        ],
        scratch_shapes=[
            pltpu.VMEM((C, 1), jnp.float32),
            pltpu.VMEM((C, 1), jnp.float32),
        ],
    )(xr)
    quantize = quant.reshape(B, C, 32, 32)
    embed_ind = ind.reshape(B, 32, 32)
    return (quantize, div[0, 0], embed_ind, perp[0, 0])


# speedup vs baseline: 1.0767x; 1.0682x over previous
import jax
import jax.numpy as jnp
from jax.experimental import pallas as pl
from jax.experimental.pallas import tpu as pltpu

B = 32
C = 768
HW = 1024
EPS = 1e-10
BLK_B = 4
GRID = B // BLK_B


def _kernel(x_ref, quant_ref, ind_ref, div_ref, perp_ref, qsum_ref, cnt_ref):
    g = pl.program_id(0)

    @pl.when(g == 0)
    def _init():
        qsum_ref[...] = jnp.zeros_like(qsum_ref)
        cnt_ref[...] = jnp.zeros_like(cnt_ref)

    qsum = qsum_ref[...]
    cnt = cnt_ref[...]
    for i in range(BLK_B):
        x = x_ref[i]
        r = jnp.maximum(x, 0.0)
        s = jnp.sum(r, axis=0, keepdims=True)
        w = 1.0 / (s + EPS)

        m = jnp.max(r, axis=0, keepdims=True)
        ci = jax.lax.broadcasted_iota(jnp.int32, (C, HW), 0)
        idx = jnp.min(jnp.where(r == m, ci, C), axis=0, keepdims=True)

        onehot = (ci == idx).astype(jnp.float32)
        quant_ref[i] = onehot
        ind_ref[i] = idx

        qsum = qsum + jnp.sum(r * w, axis=1, keepdims=True)
        cnt = cnt + jnp.sum(onehot, axis=1, keepdims=True)
    qsum_ref[...] = qsum
    cnt_ref[...] = cnt

    @pl.when(g == GRID - 1)
    def _finish():
        n = float(B * HW)
        q_bar = qsum_ref[...] / n
        div_ref[...] = jnp.mean((q_bar * C - 1.0) ** 2, keepdims=True)
        p = cnt_ref[...] / n
        perp_ref[...] = jnp.exp(-jnp.sum(p * jnp.log(p + 1e-10), keepdims=True))


@jax.jit
def kernel(x):
    xr = x.reshape(B, C, HW)
    quant, ind, div, perp = pl.pallas_call(
        _kernel,
        grid=(GRID,),
        in_specs=[pl.BlockSpec((BLK_B, C, HW), lambda g: (g, 0, 0))],
        out_specs=[
            pl.BlockSpec((BLK_B, C, HW), lambda g: (g, 0, 0)),
            pl.BlockSpec((BLK_B, 1, HW), lambda g: (g, 0, 0)),
            pl.BlockSpec((1, 1), lambda g: (0, 0)),
            pl.BlockSpec((1, 1), lambda g: (0, 0)),
        ],
        out_shape=[
            jax.ShapeDtypeStruct((B, C, HW), jnp.float32),
            jax.ShapeDtypeStruct((B, 1, HW), jnp.int32),
            jax.ShapeDtypeStruct((1, 1), jnp.float32),
            jax.ShapeDtypeStruct((1, 1), jnp.float32),
        ],
        scratch_shapes=[
            pltpu.VMEM((C, 1), jnp.float32),
            pltpu.VMEM((C, 1), jnp.float32),
        ],
    )(xr)
    quantize = quant.reshape(B, C, 32, 32)
    embed_ind = ind.reshape(B, 32, 32)
    return (quantize, div[0, 0], embed_ind, perp[0, 0])
